# baseline (speedup 1.0000x reference)
import functools

import jax
import jax.numpy as jnp
from jax import lax
from jax.experimental import pallas as pl
from jax.experimental.pallas import tpu as pltpu
from jax.experimental.pallas import tpu_sc as plsc

_TI = 1408
_NI = 2
_T = 16
_E = 16
_H = 1024


def _experts_body(x_ref, g_ref, u_ref, d_ref, eo_ref):
    i = pl.program_id(1)
    x = x_ref[...]
    g = jax.lax.dot_general(x, g_ref[0], (((1,), (1,)), ((), ())),
                            preferred_element_type=jnp.float32)
    u = jax.lax.dot_general(x, u_ref[0], (((1,), (1,)), ((), ())),
                            preferred_element_type=jnp.float32)
    h = g * jax.lax.logistic(g) * u
    contrib = jax.lax.dot_general(h, d_ref[0], (((1,), (1,)), ((), ())),
                                  preferred_element_type=jnp.float32)

    @pl.when(i == 0)
    def _init():
        eo_ref[0] = contrib

    @pl.when(i != 0)
    def _acc():
        eo_ref[0] += contrib


def _tc_experts(x, gate_w, up_w, down_w):
    t, h = x.shape
    e, i_dim, _ = gate_w.shape
    return pl.pallas_call(
        _experts_body,
        grid=(e, _NI),
        in_specs=[
            pl.BlockSpec((t, h), lambda e_, i_: (0, 0)),
            pl.BlockSpec((1, _TI, h), lambda e_, i_: (e_, i_, 0)),
            pl.BlockSpec((1, _TI, h), lambda e_, i_: (e_, i_, 0)),
            pl.BlockSpec((1, h, _TI), lambda e_, i_: (e_, 0, i_)),
        ],
        out_specs=pl.BlockSpec((1, t, h), lambda e_, i_: (e_, 0, 0)),
        out_shape=jax.ShapeDtypeStruct((e, t, h), x.dtype),
    )(x, gate_w, up_w, down_w)


def _combine_body(eo_ref, scale_ref, out_ref):
    acc = scale_ref[:, 0:1] * eo_ref[0]
    for e in range(1, _E):
        acc += scale_ref[:, e:e + 1] * eo_ref[e]
    out_ref[...] = acc


def _tc_combine(eo, scale):
    return pl.pallas_call(
        _combine_body,
        in_specs=[
            pl.BlockSpec((_E, _T, _H), lambda: (0, 0, 0)),
            pl.BlockSpec((_T, _E), lambda: (0, 0)),
        ],
        out_specs=pl.BlockSpec((_T, _H), lambda: (0, 0)),
        out_shape=jax.ShapeDtypeStruct((_T, _H), jnp.float32),
    )(eo, scale)


_sc_cache = {}


def _get_sc_router():
    if "k" in _sc_cache:
        return _sc_cache["k"]
    mesh = plsc.VectorSubcoreMesh(core_axis_name="c", subcore_axis_name="s")

    @functools.partial(
        pl.kernel,
        mesh=mesh,
        out_type=jax.ShapeDtypeStruct((_T, _E), jnp.float32),
        scratch_types=[
            pltpu.VMEM((_H,), jnp.float32),
            pltpu.VMEM((_E, _H), jnp.float32),
            pltpu.VMEM((_E,), jnp.float32),
        ],
    )
    def _sc_router(x_hbm, rw_hbm, scale_hbm, xv, wv, srow):
        cid = lax.axis_index("c")
        sid = lax.axis_index("s")
        tok = sid * 2 + cid

        @pl.when(tok < _T)
        def _():
            pltpu.sync_copy(x_hbm.at[tok], xv)
            pltpu.sync_copy(rw_hbm, wv)
            logits = []
            for e in range(_E):
                def dot_step(j, acc):
                    return acc + xv[pl.ds(16 * j, 16)] * wv[e, pl.ds(16 * j, 16)]
                acc = lax.fori_loop(0, _H // 16, dot_step,
                                    jnp.zeros((16,), jnp.float32))
                s = acc[0]
                for l in range(1, 16):
                    s = s + acc[l]
                logits.append(s)
            m1 = jnp.float32(-3.4e38)
            m2 = jnp.float32(-3.4e38)
            i1 = jnp.int32(0)
            i2 = jnp.int32(0)
            for k in range(_E):
                v = logits[k]
                is1 = v > m1
                is2 = jnp.logical_and(jnp.logical_not(is1), v > m2)
                m2 = jnp.where(is1, m1, jnp.where(is2, v, m2))
                i2 = jnp.where(is1, i1, jnp.where(is2, jnp.int32(k), i2))
                m1 = jnp.where(is1, v, m1)
                i1 = jnp.where(is1, jnp.int32(k), i1)
            eidx = lax.iota(jnp.int32, _E)
            sel = jnp.logical_or(eidx == i1, eidx == i2)
            srow[...] = jnp.where(sel, jnp.float32(0.25), jnp.float32(0.0))
            pltpu.sync_copy(srow, scale_hbm.at[tok])

    _sc_cache["k"] = _sc_router
    return _sc_router


def kernel(x, router_w, gate_w, up_w, down_w):
    scale = _get_sc_router()(x, router_w)
    eo = _tc_experts(x, gate_w, up_w, down_w)
    return _tc_combine(eo, scale)
